# baseline (speedup 1.0000x reference)
import functools

import jax
import jax.numpy as jnp
from jax import lax
from jax.experimental import pallas as pl
from jax.experimental.pallas import tpu as pltpu
from jax.experimental.pallas import tpu_sc as plsc


def _build_sc_kernel(B, L, V, D):
    info = plsc.get_sparse_core_info()
    NC, NS = info.num_cores, info.num_subcores
    NW = NC * NS
    assert B % NW == 0, (B, NW)
    seqs_per_w = B // NW
    rows_per_w = seqs_per_w * L
    n_full = L // 128
    rem = L - n_full * 128

    mesh = plsc.VectorSubcoreMesh(core_axis_name="c", subcore_axis_name="s")

    @functools.partial(
        pl.kernel,
        out_type=jax.ShapeDtypeStruct((B * L, D), jnp.float32),
        mesh=mesh,
        scratch_types=[
            pltpu.VMEM((rows_per_w,), jnp.int32),
            pltpu.VMEM((L, D), jnp.float32),
            pltpu.VMEM((L, D), jnp.float32),
        ],
        compiler_params=pltpu.CompilerParams(use_tc_tiling_on_sc=False),
    )
    def sc_kernel(seq_hbm, table_hbm, pos_hbm, out_hbm, idx_v, pos_v, rows_v):
        wid = lax.axis_index("s") * NC + lax.axis_index("c")
        base = wid * rows_per_w

        pltpu.sync_copy(seq_hbm.at[pl.ds(base, rows_per_w)], idx_v)
        pltpu.sync_copy(pos_hbm, pos_v)

        @pl.loop(0, seqs_per_w)
        def _seq_loop(s):
            off = s * L
            for c in range(n_full):
                pltpu.sync_copy(
                    table_hbm.at[idx_v.at[pl.ds(off + c * 128, 128)]],
                    rows_v.at[pl.ds(c * 128, 128)],
                )
            if rem:
                pltpu.sync_copy(
                    table_hbm.at[idx_v.at[pl.ds(off + n_full * 128, rem)]],
                    rows_v.at[pl.ds(n_full * 128, rem)],
                )

            @pl.loop(0, L)
            def _add_loop(i):
                for c in range(D // 16):
                    sl = pl.ds(c * 16, 16)
                    rows_v[i, sl] = rows_v[i, sl] + pos_v[i, sl]

            pltpu.sync_copy(rows_v, out_hbm.at[pl.ds(base + off, L)])

    return sc_kernel


def kernel(seq, token_table, pos_table):
    B, L = seq.shape
    V, D = token_table.shape
    sc = _build_sc_kernel(B, L, V, D)
    out = sc(seq.reshape(-1).astype(jnp.int32), token_table, pos_table)
    return out.reshape(B, L, D)


# speedup vs baseline: 1.0250x; 1.0250x over previous
import functools

import jax
import jax.numpy as jnp
from jax import lax
from jax.experimental import pallas as pl
from jax.experimental.pallas import tpu as pltpu
from jax.experimental.pallas import tpu_sc as plsc


def _build_sc_kernel(B, L, V, D):
    info = plsc.get_sparse_core_info()
    NC, NS = info.num_cores, info.num_subcores
    NW = NC * NS
    assert B % NW == 0, (B, NW)
    seqs_per_w = B // NW
    rows_per_w = seqs_per_w * L
    n_full = L // 128
    rem = L - n_full * 128

    mesh = plsc.VectorSubcoreMesh(core_axis_name="c", subcore_axis_name="s")

    @functools.partial(
        pl.kernel,
        out_type=jax.ShapeDtypeStruct((B * L, D), jnp.float32),
        mesh=mesh,
        scratch_types=[
            pltpu.VMEM((rows_per_w,), jnp.int32),
            pltpu.VMEM((L, D), jnp.float32),
            pltpu.VMEM((L, D), jnp.float32),
            pltpu.VMEM_SHARED((L, D), jnp.float32),
        ],
        compiler_params=pltpu.CompilerParams(use_tc_tiling_on_sc=False),
    )
    def sc_kernel(seq_hbm, table_hbm, pos_hbm, out_hbm, idx_v, pos_v, rows_v,
                  pos_sh):
        wid = lax.axis_index("s") * NC + lax.axis_index("c")
        base = wid * rows_per_w

        pltpu.sync_copy(seq_hbm.at[pl.ds(base, rows_per_w)], idx_v)
        pltpu.sync_copy(pos_hbm, pos_v)
        @pl.when(lax.axis_index("s") == 0)
        def _fill_shared():
            pltpu.sync_copy(pos_v, pos_sh)
        plsc.subcore_barrier()

        @pl.loop(0, seqs_per_w)
        def _seq_loop(s):
            off = s * L
            pltpu.sync_copy(pos_sh, rows_v)
            for c in range(n_full):
                pltpu.sync_copy(
                    table_hbm.at[idx_v.at[pl.ds(off + c * 128, 128)]],
                    rows_v.at[pl.ds(c * 128, 128)],
                    add=True,
                )
            if rem:
                pltpu.sync_copy(
                    table_hbm.at[idx_v.at[pl.ds(off + n_full * 128, rem)]],
                    rows_v.at[pl.ds(n_full * 128, rem)],
                    add=True,
                )

            pltpu.sync_copy(rows_v, out_hbm.at[pl.ds(base + off, L)])

    return sc_kernel


def kernel(seq, token_table, pos_table):
    B, L = seq.shape
    V, D = token_table.shape
    sc = _build_sc_kernel(B, L, V, D)
    out = sc(seq.reshape(-1).astype(jnp.int32), token_table, pos_table)
    return out.reshape(B, L, D)


# speedup vs baseline: 1.2204x; 1.1906x over previous
import functools

import jax
import jax.numpy as jnp
from jax import lax
from jax.experimental import pallas as pl
from jax.experimental.pallas import tpu as pltpu
from jax.experimental.pallas import tpu_sc as plsc

_NBUF = 4
_PD = 3
_GD = 2


def _build_sc_kernel(B, L, V, D):
    info = plsc.get_sparse_core_info()
    NC, NS = info.num_cores, info.num_subcores
    NW = NC * NS
    assert B % (NW * _NBUF) == 0, (B, NW)
    seqs_per_w = B // NW
    rows_per_w = seqs_per_w * L
    nchunks = seqs_per_w

    splits = []
    o = 0
    while o < L:
        n = min(128, L - o)
        splits.append((o, n))
        o += n

    mesh = plsc.VectorSubcoreMesh(core_axis_name="c", subcore_axis_name="s")

    @functools.partial(
        pl.kernel,
        out_type=jax.ShapeDtypeStruct((B * L, D), jnp.float32),
        mesh=mesh,
        scratch_types=[
            pltpu.VMEM((rows_per_w,), jnp.int32),
            pltpu.VMEM((L, D), jnp.float32),
            [pltpu.VMEM((L, D), jnp.float32) for _ in range(_NBUF)],
            pltpu.VMEM_SHARED((L, D), jnp.float32),
            pltpu.SemaphoreType.DMA((_NBUF,)),
            pltpu.SemaphoreType.DMA((_NBUF,)),
            pltpu.SemaphoreType.DMA((_NBUF,)),
        ],
        compiler_params=pltpu.CompilerParams(use_tc_tiling_on_sc=False),
    )
    def sc_kernel(seq_hbm, table_hbm, pos_hbm, out_hbm, idx_v, pos_v, bufs,
                  pos_sh, sem_p, sem_g, sem_w):
        wid = lax.axis_index("s") * NC + lax.axis_index("c")
        base = wid * rows_per_w

        pltpu.sync_copy(seq_hbm.at[pl.ds(base, rows_per_w)], idx_v)
        pltpu.sync_copy(pos_hbm, pos_v)

        @pl.when(lax.axis_index("s") == 0)
        def _fill_shared():
            pltpu.sync_copy(pos_v, pos_sh)

        plsc.subcore_barrier()

        def start_prefill(b):
            pltpu.async_copy(pos_sh, bufs[b], sem_p.at[b])

        def wait_prefill(b):
            pltpu.make_async_copy(pos_sh, bufs[b], sem_p.at[b]).wait()

        def start_gathers(c, b):
            off = c * L
            for (o, n) in splits:
                pltpu.async_copy(
                    table_hbm.at[idx_v.at[pl.ds(off + o, n)]],
                    bufs[b].at[pl.ds(o, n)],
                    sem_g.at[b],
                    add=True,
                )

        def wait_gathers(b):
            for (o, n) in splits:
                pltpu.make_async_copy(
                    table_hbm.at[idx_v.at[pl.ds(o, n)]],
                    bufs[b].at[pl.ds(o, n)],
                    sem_g.at[b],
                ).wait()

        def start_write(c, b):
            pltpu.async_copy(bufs[b], out_hbm.at[pl.ds(base + c * L, L)],
                             sem_w.at[b])

        def wait_write(b):
            pltpu.make_async_copy(bufs[b], out_hbm.at[pl.ds(base, L)],
                                  sem_w.at[b]).wait()

        for j in range(_PD):
            start_prefill(j)
        for j in range(_GD):
            wait_prefill(j)
            start_gathers(j, j)

        @pl.loop(0, nchunks // _NBUF)
        def _main(g):
            for b in range(_NBUF):
                c = g * _NBUF + b
                wait_gathers(b)
                start_write(c, b)
                bp = (b + _PD) % _NBUF

                @pl.when(c + _PD < nchunks)
                def _prefill():
                    @pl.when(c >= 1)
                    def _drain_write():
                        wait_write(bp)

                    start_prefill(bp)

                bg = (b + _GD) % _NBUF

                @pl.when(c + _GD < nchunks)
                def _gather():
                    wait_prefill(bg)
                    start_gathers(c + _GD, bg)

        for c in range(nchunks - _PD - 1, nchunks):
            wait_write(c % _NBUF)

    return sc_kernel


def kernel(seq, token_table, pos_table):
    B, L = seq.shape
    V, D = token_table.shape
    sc = _build_sc_kernel(B, L, V, D)
    out = sc(seq.reshape(-1).astype(jnp.int32), token_table, pos_table)
    return out.reshape(B, L, D)


# speedup vs baseline: 1.4926x; 1.2231x over previous
import functools

import jax
import jax.numpy as jnp
from jax import lax
from jax.experimental import pallas as pl
from jax.experimental.pallas import tpu as pltpu
from jax.experimental.pallas import tpu_sc as plsc

_NBUF = 4
_PD = 3
_GD = 2


def _build_sc_kernel(B, L, V, D):
    info = plsc.get_sparse_core_info()
    NC, NS = info.num_cores, info.num_subcores
    NW = NC * NS
    assert B % (NW * _NBUF) == 0, (B, NW)
    assert D == 128, D
    seqs_per_w = B // NW
    rows_per_w = seqs_per_w * L
    nchunks = seqs_per_w

    splits = []
    o = 0
    while o < L:
        n = min(128, L - o)
        splits.append((o, n))
        o += n

    mesh = plsc.VectorSubcoreMesh(core_axis_name="c", subcore_axis_name="s")

    @functools.partial(
        pl.kernel,
        out_type=jax.ShapeDtypeStruct((B * L, D), jnp.float32),
        mesh=mesh,
        scratch_types=[
            pltpu.VMEM((rows_per_w,), jnp.int32),
            [pltpu.VMEM((L, D), jnp.float32) for _ in range(_NBUF)],
            pltpu.VMEM_SHARED((L, D), jnp.float32),
            pltpu.SemaphoreType.DMA((_NBUF,)),
            pltpu.SemaphoreType.DMA((_NBUF,)),
            pltpu.SemaphoreType.DMA((_NBUF,)),
        ],
        compiler_params=pltpu.CompilerParams(use_tc_tiling_on_sc=True),
    )
    def sc_kernel(seq_hbm, table_hbm, pos_hbm, out_hbm, idx_v, bufs,
                  pos_sh, sem_p, sem_g, sem_w):
        wid = lax.axis_index("s") * NC + lax.axis_index("c")
        base = wid * rows_per_w

        pltpu.sync_copy(seq_hbm.at[pl.ds(base, rows_per_w)], idx_v)

        @pl.when(lax.axis_index("s") == 0)
        def _fill_shared():
            pltpu.sync_copy(pos_hbm, bufs[0])
            pltpu.sync_copy(bufs[0], pos_sh)

        plsc.subcore_barrier()

        def start_prefill(b):
            pltpu.async_copy(pos_sh, bufs[b], sem_p.at[b])

        def wait_prefill(b):
            pltpu.make_async_copy(pos_sh, bufs[b], sem_p.at[b]).wait()

        def start_gathers(c, b):
            off = c * L
            for (o, n) in splits:
                pltpu.async_copy(
                    table_hbm.at[idx_v.at[pl.ds(off + o, n)]],
                    bufs[b].at[pl.ds(o, n)],
                    sem_g.at[b],
                    add=True,
                )

        def wait_gathers(b):
            for (o, n) in splits:
                pltpu.make_async_copy(
                    table_hbm.at[idx_v.at[pl.ds(o, n)]],
                    bufs[b].at[pl.ds(o, n)],
                    sem_g.at[b],
                ).wait()

        def start_write(c, b):
            pltpu.async_copy(bufs[b], out_hbm.at[pl.ds(base + c * L, L)],
                             sem_w.at[b])

        def wait_write(b):
            pltpu.make_async_copy(bufs[b], out_hbm.at[pl.ds(base, L)],
                                  sem_w.at[b]).wait()

        for j in range(_PD):
            start_prefill(j)
        for j in range(_GD):
            wait_prefill(j)
            start_gathers(j, j)

        @pl.loop(0, nchunks // _NBUF)
        def _main(g):
            for b in range(_NBUF):
                c = g * _NBUF + b
                wait_gathers(b)
                start_write(c, b)
                bp = (b + _PD) % _NBUF

                @pl.when(c + _PD < nchunks)
                def _prefill():
                    @pl.when(c >= 1)
                    def _drain_write():
                        wait_write(bp)

                    start_prefill(bp)

                bg = (b + _GD) % _NBUF

                @pl.when(c + _GD < nchunks)
                def _gather():
                    wait_prefill(bg)
                    start_gathers(c + _GD, bg)

        for c in range(nchunks - _PD - 1, nchunks):
            wait_write(c % _NBUF)

    return sc_kernel


def kernel(seq, token_table, pos_table):
    B, L = seq.shape
    V, D = token_table.shape
    DP = 128
    tpad = jnp.pad(token_table, ((0, 0), (0, DP - D)))
    ppad = jnp.pad(pos_table, ((0, 0), (0, DP - D)))
    sc = _build_sc_kernel(B, L, V, DP)
    out = sc(seq.reshape(-1).astype(jnp.int32), tpad, ppad)
    return out[:, :D].reshape(B, L, D)


# speedup vs baseline: 1.6285x; 1.0911x over previous
import functools

import jax
import jax.numpy as jnp
from jax import lax
from jax.experimental import pallas as pl
from jax.experimental.pallas import tpu as pltpu
from jax.experimental.pallas import tpu_sc as plsc

_NBUF = 4
_PD = 3
_GD = 2


def _build_gather_kernel(B, L, V, D):
    info = plsc.get_sparse_core_info()
    NC, NS = info.num_cores, info.num_subcores
    NW = NC * NS
    assert B % (NW * _NBUF) == 0, (B, NW)
    seqs_per_w = B // NW
    rows_per_w = seqs_per_w * L
    nchunks = seqs_per_w
    DP = 128

    splits = []
    o = 0
    while o < L:
        n = min(128, L - o)
        splits.append((o, n))
        o += n

    mesh = plsc.VectorSubcoreMesh(core_axis_name="c", subcore_axis_name="s")

    @functools.partial(
        pl.kernel,
        out_type=jax.ShapeDtypeStruct((B * L, DP), jnp.float32),
        mesh=mesh,
        scratch_types=[
            pltpu.VMEM((rows_per_w,), jnp.int32),
            [pltpu.VMEM((L, D), jnp.float32) for _ in range(_NBUF)],
            pltpu.VMEM_SHARED((L, D), jnp.float32),
            pltpu.SemaphoreType.DMA((_NBUF,)),
            pltpu.SemaphoreType.DMA((_NBUF,)),
            pltpu.SemaphoreType.DMA((_NBUF,)),
        ],
        compiler_params=pltpu.CompilerParams(use_tc_tiling_on_sc=False),
    )
    def gather_k(seq_hbm, table_hbm, pos_hbm, out_hbm, idx_v, bufs,
                 pos_sh, sem_p, sem_g, sem_w):
        wid = lax.axis_index("s") * NC + lax.axis_index("c")
        base = wid * rows_per_w

        pltpu.sync_copy(seq_hbm.at[pl.ds(base, rows_per_w)], idx_v)

        @pl.when(lax.axis_index("s") == 0)
        def _fill_shared():
            pltpu.sync_copy(pos_hbm, bufs[0])
            pltpu.sync_copy(bufs[0], pos_sh)

        plsc.subcore_barrier()

        def start_prefill(b):
            pltpu.async_copy(pos_sh, bufs[b], sem_p.at[b])

        def wait_prefill(b):
            pltpu.make_async_copy(pos_sh, bufs[b], sem_p.at[b]).wait()

        def start_gathers(c, b):
            off = c * L
            for (o, n) in splits:
                pltpu.async_copy(
                    table_hbm.at[idx_v.at[pl.ds(off + o, n)]],
                    bufs[b].at[pl.ds(o, n)],
                    sem_g.at[b],
                    add=True,
                )

        def wait_gathers(b):
            for (o, n) in splits:
                pltpu.make_async_copy(
                    table_hbm.at[idx_v.at[pl.ds(o, n)]],
                    bufs[b].at[pl.ds(o, n)],
                    sem_g.at[b],
                ).wait()

        def start_write(c, b):
            pltpu.async_copy(
                bufs[b], out_hbm.at[pl.ds(base + c * L, L), pl.ds(0, D)],
                sem_w.at[b])

        def wait_write(b):
            pltpu.make_async_copy(
                bufs[b], out_hbm.at[pl.ds(base, L), pl.ds(0, D)],
                sem_w.at[b]).wait()

        for j in range(_PD):
            start_prefill(j)
        for j in range(_GD):
            wait_prefill(j)
            start_gathers(j, j)

        @pl.loop(0, nchunks // _NBUF)
        def _main(g):
            for b in range(_NBUF):
                c = g * _NBUF + b
                wait_gathers(b)
                start_write(c, b)
                bp = (b + _PD) % _NBUF

                @pl.when(c + _PD < nchunks)
                def _prefill():
                    @pl.when(c >= 1)
                    def _drain_write():
                        wait_write(bp)

                    start_prefill(bp)

                bg = (b + _GD) % _NBUF

                @pl.when(c + _GD < nchunks)
                def _gather():
                    wait_prefill(bg)
                    start_gathers(c + _GD, bg)

        for c in range(nchunks - _PD - 1, nchunks):
            wait_write(c % _NBUF)

    return gather_k


def kernel(seq, token_table, pos_table):
    B, L = seq.shape
    V, D = token_table.shape
    tflat = lax.optimization_barrier(token_table.reshape(-1))
    tbl = tflat.reshape(V, D)
    gather_k = _build_gather_kernel(B, L, V, D)
    out = gather_k(seq.reshape(-1).astype(jnp.int32), tbl, pos_table)
    return out[:, :D].reshape(B, L, D)
